# baseline (speedup 1.0000x reference)
import functools

import jax
import jax.numpy as jnp
from jax.experimental import pallas as pl
from jax.experimental.pallas import tpu as pltpu

D = 768
DFF = 512
E = 64
B = 64
NBUF = 4


def _gate_body(x_ref, gw_ref, gb_ref, amax_ref, topw_ref, rank_ref,
               counts_ref):
    x = x_ref[...]
    T = x.shape[0]
    logits = jnp.dot(x, gw_ref[...], preferred_element_type=jnp.float32)
    logits = logits + gb_ref[...]
    m = jnp.max(logits, axis=-1, keepdims=True)
    s = jnp.sum(jnp.exp(logits - m), axis=-1, keepdims=True)
    topw_ref[...] = 1.0 / s
    ids = jax.lax.broadcasted_iota(jnp.int32, logits.shape, 1)
    amax = jnp.min(jnp.where(logits == m, ids, E), axis=-1, keepdims=True)
    amax_ref[...] = amax

    hot = (ids == amax).astype(jnp.float32)
    counts = jnp.sum(hot, axis=0, keepdims=True)
    counts_ref[...] = counts.astype(jnp.int32)
    eiota_s = jax.lax.broadcasted_iota(jnp.int32, (E, E), 0)
    eiota_l = jax.lax.broadcasted_iota(jnp.int32, (E, E), 1)
    u_strict = (eiota_s < eiota_l).astype(jnp.float32)
    prefix = jnp.dot(counts, u_strict,
                     preferred_element_type=jnp.float32)
    tiota_s = jax.lax.broadcasted_iota(jnp.int32, (T, T), 0)
    tiota_l = jax.lax.broadcasted_iota(jnp.int32, (T, T), 1)
    l_strict = (tiota_s > tiota_l).astype(jnp.float32)
    earlier = jnp.dot(l_strict, hot,
                      preferred_element_type=jnp.float32)
    rank = jnp.sum(hot * (earlier + prefix), axis=1, keepdims=True)
    rank_ref[...] = rank.astype(jnp.int32)


def _moe_body(sref, x_ref, rank_ref, amaxf_ref, topw_ref,
              w1_ref, b1_ref, w2_ref, b2_ref, out_ref,
              g_scr, xb_scr, yb_scr, eidb_scr, twb_scr,
              w1buf, w2buf, sem1, sem2):
    p = pl.program_id(0)
    np_ = pl.num_programs(0)
    pb = sref[0, p]
    pe = sref[1, p]
    valid = sref[2, p]
    q = sref[3, p]
    nq = sref[5, 0]
    slot = jax.lax.rem(q, NBUF)
    prev_pb = sref[0, jnp.maximum(p - 1, 0)]
    next_pb = sref[0, jnp.minimum(p + 1, np_ - 1)]
    first_of_block = jnp.logical_or(p == 0, prev_pb != pb)
    last_of_block = jnp.logical_or(p == np_ - 1, next_pb != pb)
    prev_q = sref[3, jnp.maximum(p - 1, 0)]
    next_q = sref[3, jnp.minimum(p + 1, np_ - 1)]
    first_of_q = jnp.logical_or(p == 0, prev_q != q)
    last_of_q = jnp.logical_or(p == np_ - 1, next_q != q)

    @pl.when(p == 0)
    def _init():
        out_ref[...] = jnp.zeros_like(out_ref)

        def _prologue(i, _):
            @pl.when(i < nq)
            def _():
                ei = sref[4, i]
                pltpu.make_async_copy(w1_ref.at[ei], w1buf.at[i],
                                      sem1.at[i]).start()
                pltpu.make_async_copy(w2_ref.at[ei], w2buf.at[i],
                                      sem2.at[i]).start()
            return 0

        jax.lax.fori_loop(0, NBUF, _prologue, 0)

    @pl.when(first_of_q)
    def _wait_weights():
        pltpu.make_async_copy(w1_ref.at[pe], w1buf.at[slot],
                              sem1.at[slot]).wait()
        pltpu.make_async_copy(w2_ref.at[pe], w2buf.at[slot],
                              sem2.at[slot]).wait()

    @pl.when(first_of_block)
    def _gather():
        rank = rank_ref[...]
        lanes = jax.lax.broadcasted_iota(jnp.int32, (x_ref.shape[0], B), 1)
        gt = (rank == pb * B + lanes).astype(jnp.float32)
        g_scr[...] = gt
        dn = (((0,), (0,)), ((), ()))
        xb_scr[...] = jax.lax.dot_general(
            gt, x_ref[...], dn, preferred_element_type=jnp.float32)
        eidb_scr[...] = jax.lax.dot_general(
            gt, amaxf_ref[...], dn, preferred_element_type=jnp.float32)
        twb_scr[...] = jax.lax.dot_general(
            gt, topw_ref[...], dn, preferred_element_type=jnp.float32)
        yb_scr[...] = jnp.zeros_like(yb_scr)

    scale = jnp.where(
        jnp.logical_and(eidb_scr[...] == pe.astype(jnp.float32), valid != 0),
        twb_scr[...], 0.0)

    xb = xb_scr[...]
    h = jnp.dot(xb, w1buf[slot], preferred_element_type=jnp.float32) + b1_ref[0]
    h = 0.5 * h * (1.0 + jax.lax.erf(h * 0.7071067811865476))
    y = jnp.dot(h, w2buf[slot], preferred_element_type=jnp.float32) + b2_ref[0]
    yb_scr[...] += y * scale

    @pl.when(last_of_q)
    def _refill():
        qn = q + NBUF

        @pl.when(qn < nq)
        def _():
            en = sref[4, qn]
            sn = jax.lax.rem(qn, NBUF)
            pltpu.make_async_copy(w1_ref.at[en], w1buf.at[sn],
                                  sem1.at[sn]).start()
            pltpu.make_async_copy(w2_ref.at[en], w2buf.at[sn],
                                  sem2.at[sn]).start()

    @pl.when(last_of_block)
    def _scatter():
        out_ref[...] += jnp.dot(g_scr[...], yb_scr[...],
                                preferred_element_type=jnp.float32)


@functools.partial(jax.jit, static_argnames=())
def kernel(inputs, gate_W, gate_b, W1, b1, W2, b2):
    orig_shape = inputs.shape
    x = inputs.reshape(-1, orig_shape[-1])
    T = x.shape[0]
    NB = T // B
    MAXP = NB + E - 1
    gb2 = gate_b.reshape(1, E)
    b1r = b1.reshape(E, 1, DFF)
    b2r = b2.reshape(E, 1, D)

    amax, topw, rank, counts = pl.pallas_call(
        _gate_body,
        out_shape=(jax.ShapeDtypeStruct((T, 1), jnp.int32),
                   jax.ShapeDtypeStruct((T, 1), jnp.float32),
                   jax.ShapeDtypeStruct((T, 1), jnp.int32),
                   jax.ShapeDtypeStruct((1, E), jnp.int32)),
    )(x, gate_W, gb2)

    cum = jnp.cumsum(counts[0])
    bpos = jnp.arange(NB, dtype=jnp.int32) * B
    e_start = jnp.sum((cum[None, :] <= bpos[:, None]).astype(jnp.int32), 1)
    e_end = jnp.sum((cum[None, :] <= (bpos + B - 1)[:, None]).astype(
        jnp.int32), 1)
    num_e = e_end - e_start + 1
    offs = jnp.concatenate([jnp.zeros((1,), jnp.int32),
                            jnp.cumsum(num_e).astype(jnp.int32)])
    pids = jnp.arange(MAXP, dtype=jnp.int32)
    pair_block = jnp.clip(
        jnp.searchsorted(offs, pids, side="right").astype(jnp.int32) - 1,
        0, NB - 1)
    pair_expert = jnp.clip(
        jnp.take(e_start, pair_block) + (pids - jnp.take(offs, pair_block)),
        0, E - 1)
    pair_valid = (pids < offs[NB]).astype(jnp.int32)
    new_q = jnp.concatenate([jnp.ones((1,), jnp.bool_),
                             pair_expert[1:] != pair_expert[:-1]])
    pair_slot = jnp.cumsum(new_q.astype(jnp.int32)) - 1
    dist_expert = jnp.zeros((MAXP,), jnp.int32).at[pair_slot].set(pair_expert)
    nq = jnp.full((MAXP,), pair_slot[-1] + 1, jnp.int32)
    scalars = jnp.stack([pair_block, pair_expert, pair_valid,
                         pair_slot, dist_expert, nq])

    out = pl.pallas_call(
        _moe_body,
        grid_spec=pltpu.PrefetchScalarGridSpec(
            num_scalar_prefetch=1,
            grid=(MAXP,),
            in_specs=[
                pl.BlockSpec((T, D), lambda p, s: (0, 0)),
                pl.BlockSpec((T, 1), lambda p, s: (0, 0)),
                pl.BlockSpec((T, 1), lambda p, s: (0, 0)),
                pl.BlockSpec((T, 1), lambda p, s: (0, 0)),
                pl.BlockSpec(memory_space=pltpu.MemorySpace.HBM),
                pl.BlockSpec((1, 1, DFF), lambda p, s: (s[1, p], 0, 0)),
                pl.BlockSpec(memory_space=pltpu.MemorySpace.HBM),
                pl.BlockSpec((1, 1, D), lambda p, s: (s[1, p], 0, 0)),
            ],
            out_specs=pl.BlockSpec((T, D), lambda p, s: (0, 0)),
            scratch_shapes=[
                pltpu.VMEM((T, B), jnp.float32),
                pltpu.VMEM((B, D), jnp.float32),
                pltpu.VMEM((B, D), jnp.float32),
                pltpu.VMEM((B, 1), jnp.float32),
                pltpu.VMEM((B, 1), jnp.float32),
                pltpu.VMEM((NBUF, D, DFF), jnp.float32),
                pltpu.VMEM((NBUF, DFF, D), jnp.float32),
                pltpu.SemaphoreType.DMA((NBUF,)),
                pltpu.SemaphoreType.DMA((NBUF,)),
            ],
        ),
        out_shape=jax.ShapeDtypeStruct((T, D), jnp.float32),
        compiler_params=pltpu.CompilerParams(
            dimension_semantics=("arbitrary",),
        ),
    )(scalars, x, rank, amax.astype(jnp.float32), topw,
      W1, b1r, W2, b2r)
    return out.reshape(orig_shape)


# speedup vs baseline: 1.3966x; 1.3966x over previous
import functools

import jax
import jax.numpy as jnp
from jax.experimental import pallas as pl
from jax.experimental.pallas import tpu as pltpu

D = 768
DFF = 512
E = 64
B = 128
NBUF = 4


def _gate_body(x_ref, gw_ref, gb_ref, amax_ref, topw_ref, rank_ref,
               sc_ref):
    x = x_ref[...]
    T = x.shape[0]
    logits = jnp.dot(x, gw_ref[...], preferred_element_type=jnp.float32)
    logits = logits + gb_ref[...]
    m = jnp.max(logits, axis=-1, keepdims=True)
    s = jnp.sum(jnp.exp(logits - m), axis=-1, keepdims=True)
    topw_ref[...] = 1.0 / s
    ids = jax.lax.broadcasted_iota(jnp.int32, logits.shape, 1)
    amax = jnp.min(jnp.where(logits == m, ids, E), axis=-1, keepdims=True)
    amax_ref[...] = amax

    hot = (ids == amax).astype(jnp.float32)
    counts = jnp.sum(hot, axis=0, keepdims=True)
    eiota_s = jax.lax.broadcasted_iota(jnp.int32, (E, E), 0)
    eiota_l = jax.lax.broadcasted_iota(jnp.int32, (E, E), 1)
    u_strict = (eiota_s < eiota_l).astype(jnp.float32)
    prefix = jnp.dot(counts, u_strict,
                     preferred_element_type=jnp.float32)
    tiota_s = jax.lax.broadcasted_iota(jnp.int32, (T, T), 0)
    tiota_l = jax.lax.broadcasted_iota(jnp.int32, (T, T), 1)
    l_strict = (tiota_s > tiota_l).astype(jnp.float32)
    earlier = jnp.dot(l_strict, hot,
                      preferred_element_type=jnp.float32)
    rank = jnp.sum(hot * (earlier + prefix), axis=1, keepdims=True)
    rank_ref[...] = rank.astype(jnp.int32)

    NB = T // B
    MAXP = NB + E - 1
    cum = prefix + counts
    biota = jax.lax.broadcasted_iota(jnp.int32, (NB, 1), 0).astype(jnp.float32)
    e_start = jnp.sum((cum <= biota * B).astype(jnp.float32), 1,
                      keepdims=True)
    e_end = jnp.sum((cum <= biota * B + (B - 1)).astype(jnp.float32), 1,
                    keepdims=True)
    num_e = e_end - e_start + 1.0
    nbiota_s = jax.lax.broadcasted_iota(jnp.int32, (NB, NB), 0)
    nbiota_l = jax.lax.broadcasted_iota(jnp.int32, (NB, NB), 1)
    lnb = (nbiota_s > nbiota_l).astype(jnp.float32)
    offs = jnp.dot(lnb, num_e,
                   preferred_element_type=jnp.float32)
    total = jnp.sum(num_e)
    piota = jax.lax.broadcasted_iota(jnp.int32, (MAXP, 1), 0).astype(jnp.float32)
    pair_block = jnp.clip(
        jnp.sum((offs.reshape(1, NB) <= piota).astype(jnp.float32), 1,
                keepdims=True) - 1.0, 0.0, NB - 1.0)
    bm = (pair_block == jax.lax.broadcasted_iota(
        jnp.int32, (MAXP, NB), 1).astype(jnp.float32)).astype(jnp.float32)
    base = jnp.dot(bm, e_start - offs,
                   preferred_element_type=jnp.float32)
    pair_expert = jnp.clip(base + piota, 0.0, E - 1.0)
    pair_valid = (piota < total).astype(jnp.float32)
    eiota_row = jax.lax.broadcasted_iota(jnp.int32, (1, E), 1).astype(jnp.float32)
    pe_hot = (pair_expert == jax.lax.broadcasted_iota(
        jnp.int32, (MAXP, E), 1).astype(jnp.float32)).astype(jnp.float32)
    present = jnp.max(pe_hot, axis=0, keepdims=True)
    slot_of_e = jnp.dot(present, u_strict,
                        preferred_element_type=jnp.float32)
    pair_slot = jnp.sum(
        present * (eiota_row < pair_expert).astype(jnp.float32), 1,
        keepdims=True)
    dist_expert = jnp.sum(
        (slot_of_e == piota).astype(jnp.float32) * present * eiota_row, 1,
        keepdims=True)
    nq = jnp.sum(present) * jnp.ones_like(piota)
    zeros = jnp.zeros_like(piota)
    sc = jnp.concatenate([pair_block, pair_expert, pair_valid, pair_slot,
                          dist_expert, nq, zeros, zeros], axis=1)
    sc_ref[...] = sc.astype(jnp.int32)


def _moe_body(sref, x_ref, rank_ref, amaxf_ref, topw_ref,
              w1_ref, b1_ref, w2_ref, b2_ref, out_ref,
              g_scr, xb_scr, yb_scr, eidb_scr, twb_scr,
              w1buf, w2buf, sem1, sem2):
    p = pl.program_id(0)
    np_ = pl.num_programs(0)
    pb = sref[p, 0]
    pe = sref[p, 1]
    valid = sref[p, 2]
    q = sref[p, 3]
    nq = sref[0, 5]
    slot = jax.lax.rem(q, NBUF)
    prev_pb = sref[jnp.maximum(p - 1, 0), 0]
    next_pb = sref[jnp.minimum(p + 1, np_ - 1), 0]
    first_of_block = jnp.logical_or(p == 0, prev_pb != pb)
    last_of_block = jnp.logical_or(p == np_ - 1, next_pb != pb)
    prev_q = sref[jnp.maximum(p - 1, 0), 3]
    next_q = sref[jnp.minimum(p + 1, np_ - 1), 3]
    first_of_q = jnp.logical_or(p == 0, prev_q != q)
    last_of_q = jnp.logical_or(p == np_ - 1, next_q != q)

    @pl.when(p == 0)
    def _init():
        out_ref[...] = jnp.zeros_like(out_ref)

        def _prologue(i, _):
            @pl.when(i < nq)
            def _():
                ei = sref[i, 4]
                pltpu.make_async_copy(w1_ref.at[ei], w1buf.at[i],
                                      sem1.at[i]).start()
                pltpu.make_async_copy(w2_ref.at[ei], w2buf.at[i],
                                      sem2.at[i]).start()
            return 0

        jax.lax.fori_loop(0, NBUF, _prologue, 0)

    @pl.when(first_of_q)
    def _wait_weights():
        pltpu.make_async_copy(w1_ref.at[pe], w1buf.at[slot],
                              sem1.at[slot]).wait()
        pltpu.make_async_copy(w2_ref.at[pe], w2buf.at[slot],
                              sem2.at[slot]).wait()

    @pl.when(first_of_block)
    def _gather():
        rank = rank_ref[...]
        lanes = jax.lax.broadcasted_iota(jnp.int32, (x_ref.shape[0], B), 1)
        gt = (rank == pb * B + lanes).astype(jnp.float32)
        g_scr[...] = gt
        dn = (((0,), (0,)), ((), ()))
        xb_scr[...] = jax.lax.dot_general(
            gt, x_ref[...], dn, preferred_element_type=jnp.float32)
        eidb_scr[...] = jax.lax.dot_general(
            gt, amaxf_ref[...], dn, preferred_element_type=jnp.float32)
        twb_scr[...] = jax.lax.dot_general(
            gt, topw_ref[...], dn, preferred_element_type=jnp.float32)
        yb_scr[...] = jnp.zeros_like(yb_scr)

    scale = jnp.where(
        jnp.logical_and(eidb_scr[...] == pe.astype(jnp.float32), valid != 0),
        twb_scr[...], 0.0)

    xb = xb_scr[...]
    h = jnp.dot(xb, w1buf[slot], preferred_element_type=jnp.float32) + b1_ref[0]
    h = 0.5 * h * (1.0 + jax.lax.erf(h * 0.7071067811865476))
    y = jnp.dot(h, w2buf[slot], preferred_element_type=jnp.float32) + b2_ref[0]
    yb_scr[...] += y * scale

    @pl.when(last_of_q)
    def _refill():
        qn = q + NBUF

        @pl.when(qn < nq)
        def _():
            en = sref[qn, 4]
            sn = jax.lax.rem(qn, NBUF)
            pltpu.make_async_copy(w1_ref.at[en], w1buf.at[sn],
                                  sem1.at[sn]).start()
            pltpu.make_async_copy(w2_ref.at[en], w2buf.at[sn],
                                  sem2.at[sn]).start()

    @pl.when(last_of_block)
    def _scatter():
        out_ref[...] += jnp.dot(g_scr[...], yb_scr[...],
                                preferred_element_type=jnp.float32)


@functools.partial(jax.jit, static_argnames=())
def kernel(inputs, gate_W, gate_b, W1, b1, W2, b2):
    orig_shape = inputs.shape
    x = inputs.reshape(-1, orig_shape[-1])
    T = x.shape[0]
    NB = T // B
    MAXP = NB + E - 1
    gb2 = gate_b.reshape(1, E)
    b1r = b1.reshape(E, 1, DFF)
    b2r = b2.reshape(E, 1, D)

    amax, topw, rank, scalars = pl.pallas_call(
        _gate_body,
        out_shape=(jax.ShapeDtypeStruct((T, 1), jnp.int32),
                   jax.ShapeDtypeStruct((T, 1), jnp.float32),
                   jax.ShapeDtypeStruct((T, 1), jnp.int32),
                   jax.ShapeDtypeStruct((MAXP, 8), jnp.int32)),
    )(x, gate_W, gb2)

    out = pl.pallas_call(
        _moe_body,
        grid_spec=pltpu.PrefetchScalarGridSpec(
            num_scalar_prefetch=1,
            grid=(MAXP,),
            in_specs=[
                pl.BlockSpec((T, D), lambda p, s: (0, 0)),
                pl.BlockSpec((T, 1), lambda p, s: (0, 0)),
                pl.BlockSpec((T, 1), lambda p, s: (0, 0)),
                pl.BlockSpec((T, 1), lambda p, s: (0, 0)),
                pl.BlockSpec(memory_space=pltpu.MemorySpace.HBM),
                pl.BlockSpec((1, 1, DFF), lambda p, s: (s[p, 1], 0, 0)),
                pl.BlockSpec(memory_space=pltpu.MemorySpace.HBM),
                pl.BlockSpec((1, 1, D), lambda p, s: (s[p, 1], 0, 0)),
            ],
            out_specs=pl.BlockSpec((T, D), lambda p, s: (0, 0)),
            scratch_shapes=[
                pltpu.VMEM((T, B), jnp.float32),
                pltpu.VMEM((B, D), jnp.float32),
                pltpu.VMEM((B, D), jnp.float32),
                pltpu.VMEM((B, 1), jnp.float32),
                pltpu.VMEM((B, 1), jnp.float32),
                pltpu.VMEM((NBUF, D, DFF), jnp.float32),
                pltpu.VMEM((NBUF, DFF, D), jnp.float32),
                pltpu.SemaphoreType.DMA((NBUF,)),
                pltpu.SemaphoreType.DMA((NBUF,)),
            ],
        ),
        out_shape=jax.ShapeDtypeStruct((T, D), jnp.float32),
        compiler_params=pltpu.CompilerParams(
            dimension_semantics=("arbitrary",),
        ),
    )(scalars, x, rank, amax.astype(jnp.float32), topw,
      W1, b1r, W2, b2r)
    return out.reshape(orig_shape)
